# Initial kernel scaffold; baseline (speedup 1.0000x reference)
import jax
import jax.numpy as jnp
from jax.experimental import pallas as pl


def kernel(x):
    raise NotImplementedError("write your pallas kernel here")


# speedup vs baseline: 14.2937x; 14.2937x over previous
import jax
import jax.numpy as jnp
from jax import lax
from jax.experimental import pallas as pl
from jax.experimental.pallas import tpu as pltpu

_MAX_DET = 300
_IOU_THRES = 0.45
_CONF_THRES = 0.25
_N_BOXES = 20000
_ROWS = 160
_COLS = 128
_N_PAD = _ROWS * _COLS


def _nms_body(xt_ref, boxes_ref, cls_ref, sco_ref,
              sc_ref, y1_ref, x1_ref, y2_ref, x2_ref, area_ref,
              ms_ref, mc_ref):
    cx = xt_ref[0]
    cy = xt_ref[1]
    w = xt_ref[2]
    h = xt_ref[3]
    obj = xt_ref[4]

    y1 = cy - h / 2.0
    x1 = cx - w / 2.0
    y2 = cy + h / 2.0
    x2 = cx + w / 2.0
    y1_ref[...] = y1
    x1_ref[...] = x1
    y2_ref[...] = y2
    x2_ref[...] = x2
    area_ref[...] = (y2 - y1) * (x2 - x1)

    m0 = xt_ref[5] * obj
    a0 = jnp.zeros_like(m0)

    def cls_step(c, carry):
        m, a = carry
        s = xt_ref[5 + c] * obj
        better = s > m
        return (jnp.where(better, s, m),
                jnp.where(better, c.astype(jnp.float32), a))

    m, a = lax.fori_loop(1, 80, cls_step, (m0, a0))
    ms_ref[...] = m
    mc_ref[...] = a
    neg = jnp.float32(-jnp.inf)
    sc_ref[...] = jnp.where(obj > _CONF_THRES, m, neg)

    rowio = lax.broadcasted_iota(jnp.int32, (_ROWS, _COLS), 0)
    colio = lax.broadcasted_iota(jnp.int32, (_ROWS, _COLS), 1)
    iota = rowio * _COLS + colio
    laneio = lax.broadcasted_iota(jnp.int32, (1, _COLS), 1)

    def step(t, carry):
        sc = sc_ref[...]
        mval = jnp.max(sc)
        idx = jnp.min(jnp.where(sc == mval, iota, jnp.int32(2147483647)))
        r = idx // _COLS
        c = idx - r * _COLS
        onehot = laneio == c

        def pick(ref):
            row = ref[pl.ds(r, 1), :]
            return jnp.sum(jnp.where(onehot, row, 0.0))

        by1 = pick(y1_ref)
        bx1 = pick(x1_ref)
        by2 = pick(y2_ref)
        bx2 = pick(x2_ref)
        bs = pick(ms_ref)
        bc = pick(mc_ref)

        yy1 = jnp.maximum(by1, y1_ref[...])
        xx1 = jnp.maximum(bx1, x1_ref[...])
        yy2 = jnp.minimum(by2, y2_ref[...])
        xx2 = jnp.minimum(bx2, x2_ref[...])
        inter = jnp.maximum(yy2 - yy1, 0.0) * jnp.maximum(xx2 - xx1, 0.0)
        a1 = (by2 - by1) * (bx2 - bx1)
        iou = inter / (a1 + area_ref[...] - inter + 1e-9)
        newsc = jnp.where(iou > _IOU_THRES, neg, sc)
        newsc = jnp.where(iota == idx, neg, newsc)
        sc_ref[...] = newsc

        boxes_ref[pl.ds(t, 1), pl.ds(0, 1)] = by1.reshape(1, 1)
        boxes_ref[pl.ds(t, 1), pl.ds(1, 1)] = bx1.reshape(1, 1)
        boxes_ref[pl.ds(t, 1), pl.ds(2, 1)] = by2.reshape(1, 1)
        boxes_ref[pl.ds(t, 1), pl.ds(3, 1)] = bx2.reshape(1, 1)
        cls_ref[pl.ds(t, 1), :] = bc.reshape(1, 1)
        sco_ref[pl.ds(t, 1), :] = bs.reshape(1, 1)
        return carry

    lax.fori_loop(0, _MAX_DET, step, 0)


@jax.jit
def kernel(x):
    p = x[0]
    pad = jnp.zeros((_N_PAD - _N_BOXES, 85), jnp.float32)
    xp = jnp.concatenate([p, pad], axis=0)
    xt = xp.T.reshape(85, _ROWS, _COLS)
    boxes, cls, sco = pl.pallas_call(
        _nms_body,
        out_shape=[
            jax.ShapeDtypeStruct((_MAX_DET, 4), jnp.float32),
            jax.ShapeDtypeStruct((_MAX_DET, 1), jnp.float32),
            jax.ShapeDtypeStruct((_MAX_DET, 1), jnp.float32),
        ],
        scratch_shapes=[pltpu.VMEM((_ROWS, _COLS), jnp.float32)] * 8,
    )(xt)
    return boxes[None], cls[:, 0][None], sco[:, 0][None]
